# Initial kernel scaffold; baseline (speedup 1.0000x reference)
import jax
import jax.numpy as jnp
from jax.experimental import pallas as pl


def kernel(node_features, edge_index, W_msg, b_msg, W_upd, b_upd, gamma, beta):
    raise NotImplementedError("write your pallas kernel here")


# speedup vs baseline: 1.6353x; 1.6353x over previous
import functools

import jax
import jax.numpy as jnp
from jax import lax
from jax.experimental import pallas as pl
from jax.experimental.pallas import tpu as pltpu
from jax.experimental.pallas import tpu_sc as plsc

N = 10000
D = 128
H = D // 2
E = 320000

NP = 10240
OWN = 5000
NA = 5120
CHUNK = 128
CPT = 160
EPT = CPT * CHUNK
E_PAD = 16 * EPT
ROWS_PER_TILE = NA // 16
NBLK = 1024
PBLK = 1000


def _pre_body(x_ref, wm_ref, bm_ref, a_ref, b_ref):
    x = x_ref[...]
    dn = (((1,), (1,)), ((), ()))
    a_ref[0] = lax.dot_general(x, wm_ref[:, :D], dn,
                               preferred_element_type=jnp.float32) + bm_ref[0]
    b_ref[0] = lax.dot_general(x, wm_ref[:, D:], dn,
                               preferred_element_type=jnp.float32)


_pre_call = pl.pallas_call(
    _pre_body,
    grid=(2, NP // NBLK),
    in_specs=[
        pl.BlockSpec((NBLK, D), lambda h, i: (i, 0)),
        pl.BlockSpec((H, 2 * D), lambda h, i: (h, 0)),
        pl.BlockSpec((1, 1, H), lambda h, i: (h, 0, 0)),
    ],
    out_specs=[
        pl.BlockSpec((1, NBLK, H), lambda h, i: (h, i, 0)),
        pl.BlockSpec((1, NBLK, H), lambda h, i: (h, i, 0)),
    ],
    out_shape=[
        jax.ShapeDtypeStruct((2, NP, H), jnp.float32),
        jax.ShapeDtypeStruct((2, NP, H), jnp.float32),
    ],
)


_mesh = plsc.VectorSubcoreMesh(core_axis_name="c", subcore_axis_name="s")


@functools.partial(
    pl.kernel,
    mesh=_mesh,
    compiler_params=pltpu.CompilerParams(use_tc_tiling_on_sc=False),
    out_type=[
        jax.ShapeDtypeStruct((2, 2, NA, H), jnp.float32),
        jax.ShapeDtypeStruct((2, NA, 16), jnp.float32),
    ],
    scratch_types=[
        pltpu.VMEM((CPT, CHUNK), jnp.int32),
        pltpu.VMEM((CPT, CHUNK), jnp.int32),
        pltpu.VMEM((CPT, CHUNK), jnp.int32),
        pltpu.VMEM((CHUNK, H), jnp.float32),
        pltpu.VMEM((CHUNK, H), jnp.float32),
        pltpu.VMEM((CHUNK, H), jnp.float32),
        pltpu.VMEM((CHUNK, 16), jnp.float32),
        pltpu.VMEM_SHARED((NA, H), jnp.float32),
        pltpu.VMEM_SHARED((NA, 16), jnp.float32),
        pltpu.SemaphoreType.DMA,
    ],
)
def _sc_call(a_hbm, b_hbm, src_hbm, tgt_hbm, msg_out, cnt_out,
             src_adj, tgt_adj, loc_idx, bufa, bufb, msgb, cntb,
             acc_msg, acc_cnt, sem):
    cid = lax.axis_index("c")
    sid = lax.axis_index("s")
    row0 = sid * ROWS_PER_TILE

    zero16 = jnp.zeros((16,), jnp.float32)
    onehot = jnp.where(lax.iota(jnp.int32, 16) == 0, 1.0, 0.0).astype(jnp.float32)

    pltpu.sync_copy(src_hbm.at[pl.ds(sid * CPT, CPT)], src_adj)
    pltpu.sync_copy(tgt_hbm.at[pl.ds(sid * CPT, CPT)], tgt_adj)
    cofs = cid * NP

    def bias_body(r, carry):
        for j in range(CHUNK // 16):
            sl = pl.ds(j * 16, 16)
            src_adj[r, sl] = src_adj[r, sl] + cofs
            tgt_adj[r, sl] = tgt_adj[r, sl] + cofs
        return carry

    lax.fori_loop(0, CPT, bias_body, 0)

    for phase in range(2):
        def zbody(e, carry):
            for j in range(H // 16):
                msgb[e, pl.ds(j * 16, 16)] = zero16
            cntb[e, :] = zero16
            return carry

        lax.fori_loop(0, CHUNK, zbody, 0)
        for t in range(ROWS_PER_TILE // CHUNK):
            pltpu.sync_copy(msgb, acc_msg.at[pl.ds(row0 + t * CHUNK, CHUNK)])
            pltpu.sync_copy(cntb, acc_cnt.at[pl.ds(row0 + t * CHUNK, CHUNK)])
        rem = ROWS_PER_TILE % CHUNK
        if rem:
            base = row0 + (ROWS_PER_TILE // CHUNK) * CHUNK
            pltpu.sync_copy(msgb.at[pl.ds(0, rem)],
                            acc_msg.at[pl.ds(base, rem)])
            pltpu.sync_copy(cntb.at[pl.ds(0, rem)],
                            acc_cnt.at[pl.ds(base, rem)])

        def obody(e, carry):
            cntb[e, :] = onehot
            return carry

        lax.fori_loop(0, CHUNK, obody, 0)
        lo = cofs + phase * OWN

        def loc_body(r, carry):
            for j in range(CHUNK // 16):
                sl = pl.ds(j * 16, 16)
                lt = tgt_adj[r, sl] - lo
                valid = (lt >= 0) & (lt < OWN)
                loc_idx[r, sl] = jnp.where(valid, lt, OWN)
            return carry

        lax.fori_loop(0, CPT, loc_body, 0)
        plsc.subcore_barrier()

        def chunk_body(i, carry):
            pltpu.async_copy(a_hbm.at[src_adj.at[i]], bufa, sem).wait()
            pltpu.async_copy(b_hbm.at[tgt_adj.at[i]], bufb, sem).wait()

            def ebody(e, c2):
                for j in range(H // 16):
                    sl = pl.ds(j * 16, 16)
                    msgb[e, sl] = jnp.maximum(bufa[e, sl] + bufb[e, sl], 0.0)
                return c2

            lax.fori_loop(0, CHUNK, ebody, 0)
            pltpu.sync_copy(msgb, acc_msg.at[loc_idx.at[i]], add=True)

            @pl.when(cid == 0)
            def _():
                pltpu.sync_copy(cntb, acc_cnt.at[loc_idx.at[i]], add=True)

            return carry

        lax.fori_loop(0, CPT, chunk_body, 0)
        plsc.subcore_barrier()

        pltpu.sync_copy(acc_msg.at[pl.ds(row0, ROWS_PER_TILE)],
                        msg_out.at[cid, phase, pl.ds(row0, ROWS_PER_TILE)])

        @pl.when(cid == 0)
        def _():
            pltpu.sync_copy(acc_cnt.at[pl.ds(row0, ROWS_PER_TILE)],
                            cnt_out.at[phase, pl.ds(row0, ROWS_PER_TILE)])

        plsc.subcore_barrier()


def _post_body(x_ref, mp_ref, cp_ref, wu_ref, bu_ref, g_ref, bt_ref, o_ref):
    x = x_ref[...]
    cnt = jnp.maximum(cp_ref[0][:, 0:1], 1.0)
    m0 = mp_ref[0, 0] / cnt
    m1 = mp_ref[1, 0] / cnt
    dn = (((1,), (1,)), ((), ()))
    upd = (lax.dot_general(x, wu_ref[:, :D], dn, preferred_element_type=jnp.float32)
           + lax.dot_general(m0, wu_ref[:, D:D + H], dn,
                             preferred_element_type=jnp.float32)
           + lax.dot_general(m1, wu_ref[:, D + H:], dn,
                             preferred_element_type=jnp.float32)
           + bu_ref[...])
    h = jnp.maximum(upd, 0.0) + x
    mean = jnp.mean(h, axis=-1, keepdims=True)
    var = jnp.mean((h - mean) * (h - mean), axis=-1, keepdims=True)
    o_ref[...] = (h - mean) / jnp.sqrt(var + 1e-5) * g_ref[...] + bt_ref[...]


_post_call = pl.pallas_call(
    _post_body,
    grid=(N // PBLK,),
    in_specs=[
        pl.BlockSpec((PBLK, D), lambda i: (i, 0)),
        pl.BlockSpec((2, 1, PBLK, D // 2), lambda i: (0, i // 5, i % 5, 0)),
        pl.BlockSpec((1, PBLK, 16), lambda i: (i // 5, i % 5, 0)),
        pl.BlockSpec((D, 2 * D), lambda i: (0, 0)),
        pl.BlockSpec((1, D), lambda i: (0, 0)),
        pl.BlockSpec((1, D), lambda i: (0, 0)),
        pl.BlockSpec((1, D), lambda i: (0, 0)),
    ],
    out_specs=pl.BlockSpec((PBLK, D), lambda i: (i, 0)),
    out_shape=jax.ShapeDtypeStruct((N, D), jnp.float32),
)


def kernel(node_features, edge_index, W_msg, b_msg, W_upd, b_upd, gamma, beta):
    xp = jnp.pad(node_features, ((0, NP - N), (0, 0)))
    a, b = _pre_call(xp, W_msg, b_msg.reshape(2, 1, H))
    a = a.reshape(2 * NP, H)
    b = b.reshape(2 * NP, H)

    src_p = jnp.pad(edge_index[0], (0, E_PAD - E)).reshape(16 * CPT, CHUNK)
    tgt_p = jnp.pad(edge_index[1], (0, E_PAD - E),
                    constant_values=N).reshape(16 * CPT, CHUNK)

    msg_part, cnt_part = _sc_call(a, b, src_p, tgt_p)

    return _post_call(node_features, msg_part, cnt_part, W_upd,
                      b_upd.reshape(1, D), gamma.reshape(1, D),
                      beta.reshape(1, D))
